# Initial kernel scaffold; baseline (speedup 1.0000x reference)
import jax
import jax.numpy as jnp
from jax.experimental import pallas as pl


def kernel(x, edge_index, Wl1, bl1, Wr1, Wl2, bl2, Wr2):
    raise NotImplementedError("write your pallas kernel here")


# speedup vs baseline: 6.6536x; 6.6536x over previous
import functools

import jax
import jax.numpy as jnp
from jax import lax
from jax.experimental import pallas as pl
from jax.experimental.pallas import tpu as pltpu
from jax.experimental.pallas import tpu_sc as plsc

N = 10000
D = 128
E = 320000

NC = 2
NS = 16
NW = NC * NS

CH = 128
NCH = 80
EPT = CH * NCH
EPAD = EPT * NW
NPADROWS = 240
NACC = N + NPADROWS
RPT = NACC // NS
IB = 4
NIB = NCH // IB

_DOT = dict(preferred_element_type=jnp.float32, precision=lax.Precision.HIGHEST)


def _xwt(a, w):
    return lax.dot_general(a, w, (((1,), (1,)), ((), ())), **_DOT)


def _sc_body(table, srcg, dstg, zacc, agg_out,
             src_v, dst_v, rows_v, acc_sh, sem):
    c = lax.axis_index("c")
    s = lax.axis_index("s")
    wid = c * NS + s
    lo = s * RPT

    pltpu.sync_copy(zacc.at[pl.ds(lo, RPT)], acc_sh.at[pl.ds(lo, RPT)])
    plsc.subcore_barrier()

    def outer(jb, carry):
        ib0 = wid * NCH + jb * IB
        pltpu.sync_copy(srcg.at[pl.ds(ib0, IB)], src_v)
        pltpu.sync_copy(dstg.at[pl.ds(ib0, IB)], dst_v)

        def inner(j, c2):
            pltpu.async_copy(table.at[src_v.at[j]], rows_v, sem).wait()
            pltpu.sync_copy(rows_v, acc_sh.at[dst_v.at[j]], add=True)
            return c2

        return lax.fori_loop(0, IB, inner, carry)

    lax.fori_loop(0, NIB, outer, 0)
    plsc.subcore_barrier()

    dlo = c * NACC + lo
    pltpu.sync_copy(acc_sh.at[pl.ds(lo, RPT)], agg_out.at[pl.ds(dlo, RPT)])


def _cnt_body(dstg, zcnt, ones, cnt_out, dst_v, ones_v, cnt_sh):
    c = lax.axis_index("c")
    s = lax.axis_index("s")
    wid = c * NS + s
    lo = s * RPT

    pltpu.sync_copy(zcnt.at[pl.ds(lo, RPT)], cnt_sh.at[pl.ds(lo, RPT)])
    pltpu.sync_copy(ones, ones_v)
    plsc.subcore_barrier()

    def outer(jb, carry):
        ib0 = wid * NCH + jb * IB
        pltpu.sync_copy(dstg.at[pl.ds(ib0, IB)], dst_v)

        def inner(j, c2):
            pltpu.sync_copy(ones_v, cnt_sh.at[dst_v.at[j]], add=True)
            return c2

        return lax.fori_loop(0, IB, inner, carry)

    lax.fori_loop(0, NIB, outer, 0)
    plsc.subcore_barrier()

    dlo = c * NACC + lo
    pltpu.sync_copy(cnt_sh.at[pl.ds(lo, RPT)], cnt_out.at[pl.ds(dlo, RPT)])


_sc_cnt = pl.kernel(
    _cnt_body,
    out_type=jax.ShapeDtypeStruct((NC * NACC, D), jnp.float32),
    mesh=plsc.VectorSubcoreMesh(core_axis_name="c", subcore_axis_name="s"),
    scratch_types=[
        pltpu.VMEM((IB, CH), jnp.int32),
        pltpu.VMEM((CH, D), jnp.float32),
        pltpu.VMEM_SHARED((NACC, D), jnp.float32),
    ],
)


_sc_seg = pl.kernel(
    _sc_body,
    out_type=jax.ShapeDtypeStruct((NC * NACC, D), jnp.float32),
    mesh=plsc.VectorSubcoreMesh(core_axis_name="c", subcore_axis_name="s"),
    scratch_types=[
        pltpu.VMEM((IB, CH), jnp.int32),
        pltpu.VMEM((IB, CH), jnp.int32),
        pltpu.VMEM((CH, D), jnp.float32),
        pltpu.VMEM_SHARED((NACC, D), jnp.float32),
        pltpu.SemaphoreType.DMA,
    ],
)


_BR = 1000


def _mmt_body(x_ref, w_ref, o_ref):
    o_ref[...] = _xwt(x_ref[...], w_ref[...])


def _mmt(x, w):
    return pl.pallas_call(
        _mmt_body,
        grid=(N // _BR,),
        in_specs=[
            pl.BlockSpec((_BR, D), lambda i: (i, 0)),
            pl.BlockSpec((D, D), lambda i: (0, 0)),
        ],
        out_specs=pl.BlockSpec((_BR, D), lambda i: (i, 0)),
        out_shape=jax.ShapeDtypeStruct((N, D), jnp.float32),
    )(x, w)


def _mid_body(a0, a1, c0, c1, x_ref, wr, b, wl2, h_ref, y2_ref):
    cnt = (c0[...] + c1[...])[:, 0:1]
    inv = 1.0 / jnp.maximum(cnt, 1.0)
    mean = (a0[...] + a1[...]) * inv
    h = mean + b[...] + _xwt(x_ref[...], wr[...])
    h = jnp.maximum(h, 0.0)
    h_ref[...] = h
    y2_ref[...] = _xwt(h, wl2[...])


def _mid(a0, a1, c0, c1, x, wr, b, wl2):
    blk = pl.BlockSpec((_BR, D), lambda i: (i, 0))
    cblk = pl.BlockSpec((_BR, 16), lambda i: (i, 0))
    wblk = pl.BlockSpec((D, D), lambda i: (0, 0))
    bblk = pl.BlockSpec((1, D), lambda i: (0, 0))
    return pl.pallas_call(
        _mid_body,
        grid=(N // _BR,),
        in_specs=[blk, blk, cblk, cblk, blk, wblk, bblk, wblk],
        out_specs=[blk, blk],
        out_shape=[
            jax.ShapeDtypeStruct((N, D), jnp.float32),
            jax.ShapeDtypeStruct((N, D), jnp.float32),
        ],
    )(a0, a1, c0, c1, x, wr, b, wl2)


def _fin_body(a0, a1, c0, c1, h_ref, wr, b, o_ref):
    cnt = (c0[...] + c1[...])[:, 0:1]
    inv = 1.0 / jnp.maximum(cnt, 1.0)
    mean = (a0[...] + a1[...]) * inv
    o_ref[...] = mean + b[...] + _xwt(h_ref[...], wr[...])


def _fin(a0, a1, c0, c1, h, wr, b):
    blk = pl.BlockSpec((_BR, D), lambda i: (i, 0))
    cblk = pl.BlockSpec((_BR, 16), lambda i: (i, 0))
    wblk = pl.BlockSpec((D, D), lambda i: (0, 0))
    bblk = pl.BlockSpec((1, D), lambda i: (0, 0))
    return pl.pallas_call(
        _fin_body,
        grid=(N // _BR,),
        in_specs=[blk, blk, cblk, cblk, blk, wblk, bblk],
        out_specs=blk,
        out_shape=jax.ShapeDtypeStruct((N, D), jnp.float32),
    )(a0, a1, c0, c1, h, wr, b)


def kernel(x, edge_index, Wl1, bl1, Wr1, Wl2, bl2, Wr2):
    src = edge_index[0].astype(jnp.int32)
    dst = edge_index[1].astype(jnp.int32)
    padn = EPAD - E
    ar = jnp.arange(padn, dtype=jnp.int32)
    pad_src = (ar * 37) % N
    pad_dst = N + (ar % NPADROWS)
    srcg = jnp.concatenate([src, pad_src]).reshape(NW * NCH, CH)
    dstg = jnp.concatenate([dst, pad_dst]).reshape(NW * NCH, CH)
    zacc = jnp.zeros((NACC, D), jnp.float32)
    bl1r = bl1.reshape(1, D)
    bl2r = bl2.reshape(1, D)

    ones = jnp.ones((CH, D), jnp.float32)
    cntf = _sc_cnt(dstg, zacc, ones)
    c0, c1 = cntf[:N, :16], cntf[NACC:NACC + N, :16]

    y1 = _mmt(x, Wl1)
    aggf = _sc_seg(y1, srcg, dstg, zacc)
    a0, a1 = aggf[:N], aggf[NACC:NACC + N]
    h, y2 = _mid(a0, a1, c0, c1, x, Wr1, bl1r, Wl2)
    aggf2 = _sc_seg(y2, srcg, dstg, zacc)
    b0, b1 = aggf2[:N], aggf2[NACC:NACC + N]
    return _fin(b0, b1, c0, c1, h, Wr2, bl2r)


# speedup vs baseline: 8.9803x; 1.3497x over previous
import functools

import jax
import jax.numpy as jnp
from jax import lax
from jax.experimental import pallas as pl
from jax.experimental.pallas import tpu as pltpu
from jax.experimental.pallas import tpu_sc as plsc

N = 10000
D = 128
E = 320000

NC = 2
NS = 16
NW = NC * NS

CH = 128
NCH = 80
EPT = CH * NCH
EPAD = EPT * NW
NPADROWS = 240
NACC = N + NPADROWS
RPT = NACC // NS
IB = 8
NIB = NCH // IB

_DOT = dict(preferred_element_type=jnp.float32, precision=lax.Precision.HIGHEST)


def _xwt(a, w):
    return lax.dot_general(a, w, (((1,), (1,)), ((), ())), **_DOT)


def _sc_body(table, srcg, dstg, zacc, agg_out,
             src_v, dst_v, rows_a, rows_b, acc_sh, sem_a, sem_b):
    c = lax.axis_index("c")
    s = lax.axis_index("s")
    wid = c * NS + s
    lo = s * RPT

    pltpu.sync_copy(zacc.at[pl.ds(lo, RPT)], acc_sh.at[pl.ds(lo, RPT)])
    plsc.subcore_barrier()

    bufs = (rows_a, rows_b)
    sems = (sem_a, sem_b)

    def outer(jb, carry):
        ib0 = wid * NCH + jb * IB
        pltpu.sync_copy(srcg.at[pl.ds(ib0, IB)], src_v)
        pltpu.sync_copy(dstg.at[pl.ds(ib0, IB)], dst_v)

        h = [None] * IB
        h[0] = pltpu.async_copy(table.at[src_v.at[0]], bufs[0], sems[0])
        for j in range(1, IB + 1):
            if j < IB:
                h[j] = pltpu.async_copy(table.at[src_v.at[j]], bufs[j % 2],
                                        sems[j % 2])
            h[j - 1].wait()
            pltpu.sync_copy(bufs[(j - 1) % 2], acc_sh.at[dst_v.at[j - 1]],
                            add=True)
        return carry

    lax.fori_loop(0, NIB, outer, 0)
    plsc.subcore_barrier()

    dlo = c * NACC + lo
    pltpu.sync_copy(acc_sh.at[pl.ds(lo, RPT)], agg_out.at[pl.ds(dlo, RPT)])


def _cnt_body(dstg, zcnt, ones, cnt_out, dst_v, ones_v, cnt_sh):
    c = lax.axis_index("c")
    s = lax.axis_index("s")
    wid = c * NS + s
    lo = s * RPT

    pltpu.sync_copy(zcnt.at[pl.ds(lo, RPT)], cnt_sh.at[pl.ds(lo, RPT)])
    pltpu.sync_copy(ones, ones_v)
    plsc.subcore_barrier()

    def outer(jb, carry):
        ib0 = wid * NCH + jb * IB
        pltpu.sync_copy(dstg.at[pl.ds(ib0, IB)], dst_v)

        def inner(j, c2):
            pltpu.sync_copy(ones_v, cnt_sh.at[dst_v.at[j]], add=True)
            return c2

        return lax.fori_loop(0, IB, inner, carry)

    lax.fori_loop(0, NIB, outer, 0)
    plsc.subcore_barrier()

    dlo = c * NACC + lo
    pltpu.sync_copy(cnt_sh.at[pl.ds(lo, RPT)], cnt_out.at[pl.ds(dlo, RPT)])


_sc_cnt = pl.kernel(
    _cnt_body,
    out_type=jax.ShapeDtypeStruct((NC * NACC, D), jnp.float32),
    mesh=plsc.VectorSubcoreMesh(core_axis_name="c", subcore_axis_name="s"),
    scratch_types=[
        pltpu.VMEM((IB, CH), jnp.int32),
        pltpu.VMEM((CH, D), jnp.float32),
        pltpu.VMEM_SHARED((NACC, D), jnp.float32),
    ],
)


_sc_seg = pl.kernel(
    _sc_body,
    out_type=jax.ShapeDtypeStruct((NC * NACC, D), jnp.float32),
    mesh=plsc.VectorSubcoreMesh(core_axis_name="c", subcore_axis_name="s"),
    scratch_types=[
        pltpu.VMEM((IB, CH), jnp.int32),
        pltpu.VMEM((IB, CH), jnp.int32),
        pltpu.VMEM((CH, D), jnp.float32),
        pltpu.VMEM((CH, D), jnp.float32),
        pltpu.VMEM_SHARED((NACC, D), jnp.float32),
        pltpu.SemaphoreType.DMA,
        pltpu.SemaphoreType.DMA,
    ],
)


_BR = 1000


def _mmt_body(x_ref, w_ref, o_ref):
    o_ref[...] = _xwt(x_ref[...], w_ref[...])


def _mmt(x, w):
    return pl.pallas_call(
        _mmt_body,
        grid=(N // _BR,),
        in_specs=[
            pl.BlockSpec((_BR, D), lambda i: (i, 0)),
            pl.BlockSpec((D, D), lambda i: (0, 0)),
        ],
        out_specs=pl.BlockSpec((_BR, D), lambda i: (i, 0)),
        out_shape=jax.ShapeDtypeStruct((N, D), jnp.float32),
    )(x, w)


def _mid_body(a0, a1, c0, c1, x_ref, wr, b, wl2, h_ref, y2_ref):
    cnt = (c0[...] + c1[...])[:, 0:1]
    inv = 1.0 / jnp.maximum(cnt, 1.0)
    mean = (a0[...] + a1[...]) * inv
    h = mean + b[...] + _xwt(x_ref[...], wr[...])
    h = jnp.maximum(h, 0.0)
    h_ref[...] = h
    y2_ref[...] = _xwt(h, wl2[...])


def _mid(a0, a1, c0, c1, x, wr, b, wl2):
    blk = pl.BlockSpec((_BR, D), lambda i: (i, 0))
    cblk = pl.BlockSpec((_BR, 16), lambda i: (i, 0))
    wblk = pl.BlockSpec((D, D), lambda i: (0, 0))
    bblk = pl.BlockSpec((1, D), lambda i: (0, 0))
    return pl.pallas_call(
        _mid_body,
        grid=(N // _BR,),
        in_specs=[blk, blk, cblk, cblk, blk, wblk, bblk, wblk],
        out_specs=[blk, blk],
        out_shape=[
            jax.ShapeDtypeStruct((N, D), jnp.float32),
            jax.ShapeDtypeStruct((N, D), jnp.float32),
        ],
    )(a0, a1, c0, c1, x, wr, b, wl2)


def _fin_body(a0, a1, c0, c1, h_ref, wr, b, o_ref):
    cnt = (c0[...] + c1[...])[:, 0:1]
    inv = 1.0 / jnp.maximum(cnt, 1.0)
    mean = (a0[...] + a1[...]) * inv
    o_ref[...] = mean + b[...] + _xwt(h_ref[...], wr[...])


def _fin(a0, a1, c0, c1, h, wr, b):
    blk = pl.BlockSpec((_BR, D), lambda i: (i, 0))
    cblk = pl.BlockSpec((_BR, 16), lambda i: (i, 0))
    wblk = pl.BlockSpec((D, D), lambda i: (0, 0))
    bblk = pl.BlockSpec((1, D), lambda i: (0, 0))
    return pl.pallas_call(
        _fin_body,
        grid=(N // _BR,),
        in_specs=[blk, blk, cblk, cblk, blk, wblk, bblk],
        out_specs=blk,
        out_shape=jax.ShapeDtypeStruct((N, D), jnp.float32),
    )(a0, a1, c0, c1, h, wr, b)


def kernel(x, edge_index, Wl1, bl1, Wr1, Wl2, bl2, Wr2):
    src = edge_index[0].astype(jnp.int32)
    dst = edge_index[1].astype(jnp.int32)
    padn = EPAD - E
    ar = jnp.arange(padn, dtype=jnp.int32)
    pad_src = (ar * 37) % N
    pad_dst = N + (ar % NPADROWS)
    srcg = jnp.concatenate([src, pad_src]).reshape(NW * NCH, CH)
    dstg = jnp.concatenate([dst, pad_dst]).reshape(NW * NCH, CH)
    zacc = jnp.zeros((NACC, D), jnp.float32)
    bl1r = bl1.reshape(1, D)
    bl2r = bl2.reshape(1, D)

    ones = jnp.ones((CH, D), jnp.float32)
    cntf = _sc_cnt(dstg, zacc, ones)
    c0, c1 = cntf[:N, :16], cntf[NACC:NACC + N, :16]

    y1 = _mmt(x, Wl1)
    aggf = _sc_seg(y1, srcg, dstg, zacc)
    a0, a1 = aggf[:N], aggf[NACC:NACC + N]
    h, y2 = _mid(a0, a1, c0, c1, x, Wr1, bl1r, Wl2)
    aggf2 = _sc_seg(y2, srcg, dstg, zacc)
    b0, b1 = aggf2[:N], aggf2[NACC:NACC + N]
    return _fin(b0, b1, c0, c1, h, Wr2, bl2r)
